# baseline (speedup 1.0000x reference)
import functools

import jax
import jax.numpy as jnp
from jax import lax
from jax.experimental import pallas as pl
from jax.experimental.pallas import tpu as pltpu
from jax.experimental.pallas import tpu_sc as plsc

_R = 8
_CAP_E = 8192
_CAP_B = _CAP_E + 64


@functools.cache
def _build(n: int, v: int, d: int):
    info = plsc.get_sparse_core_info()
    nc, ns = info.num_cores, info.num_subcores
    nw = nc * ns
    assert v % nw == 0 and (v // nw) % _R == 0 and n % 16 == 0
    assert n <= (1 << 16) and (v // nw) <= (1 << 8)
    span = v // nw
    n_sub = span // _R
    n_idx_chunks = 16
    chunk_len = n // n_idx_chunks
    assert chunk_len % 16 == 0

    mesh = plsc.VectorSubcoreMesh(core_axis_name="c", subcore_axis_name="s")

    @functools.partial(
        pl.kernel,
        out_type=jax.ShapeDtypeStruct((n, d), jnp.float32),
        mesh=mesh,
        compiler_params=pltpu.CompilerParams(needs_layout_passes=False),
        scratch_types=[
            pltpu.VMEM((2, chunk_len), jnp.int32),
            pltpu.VMEM((_CAP_B,), jnp.int32),
            pltpu.VMEM((2 * _CAP_B,), jnp.int32),
            pltpu.VMEM((_R, d), jnp.float32),
            pltpu.VMEM((16,), jnp.int32),
            pltpu.SemaphoreType.DMA,
            pltpu.SemaphoreType.DMA,
            pltpu.SemaphoreType.DMA,
            pltpu.SemaphoreType.DMA,
        ],
    )
    def body(idx_hbm, table_hbm, out_hbm, idxbuf, e_pack, lbuf,
             rows, ovf_key, sem_idx0, sem_idx1, sem_row, sem_out):
        sem_idx = (sem_idx0, sem_idx1)
        wid = lax.axis_index("s") * nc + lax.axis_index("c")
        lo = wid * span
        hi = lo + span
        lane = lax.iota(jnp.int32, 16)

        def lane_at(vec, j):
            return jnp.sum(jnp.where(lane == j, vec, 0))

        def drain_outs(cnt):
            def w(_, c):
                pltpu.make_async_copy(
                    rows.at[0], out_hbm.at[0], sem_out).wait()
                return c
            lax.fori_loop(0, cnt, w, 0)

        def wait_rows():
            pltpu.make_async_copy(
                table_hbm.at[pl.ds(0, _R)], rows, sem_row).wait()

        def do_overflow(kv, posv, m):
            novf = jnp.sum(m.astype(jnp.int32))

            @pl.when(novf > 0)
            def _():
                ovf_key[pl.ds(0, 16)] = jnp.zeros((16,), jnp.int32)
                pf = plsc.cumsum(m.astype(jnp.int32))
                dst = jnp.where(m, pf - 1, 0)
                plsc.store_scatter(ovf_key, [dst], kv, mask=m)
                for b in range(2):
                    @pl.when(novf > 8 * b)
                    def _():
                        pltpu.async_copy(
                            table_hbm.at[ovf_key.at[pl.ds(8 * b, 8)]],
                            rows, sem_row)
                        wait_rows()
                        cb = jnp.minimum(novf - 8 * b, 8)

                        def issue(j, c):
                            srcm = m & (jnp.where(m, pf - 1, -1)
                                        == (8 * b + j))
                            pp = jnp.sum(jnp.where(srcm, posv, 0))
                            pltpu.async_copy(
                                rows.at[j], out_hbm.at[pp], sem_out)
                            return c
                        lax.fori_loop(0, cb, issue, 0)
                        drain_outs(cb)

        pltpu.sync_copy(idx_hbm.at[pl.ds(0, chunk_len)], idxbuf.at[0])
        cursor = jnp.int32(0)
        for ch in range(n_idx_chunks):
            if ch + 1 < n_idx_chunks:
                pltpu.async_copy(
                    idx_hbm.at[pl.ds((ch + 1) * chunk_len, chunk_len)],
                    idxbuf.at[(ch + 1) % 2], sem_idx[(ch + 1) % 2])
            if ch > 0:
                pltpu.make_async_copy(
                    idx_hbm.at[pl.ds(0, chunk_len)],
                    idxbuf.at[ch % 2], sem_idx[ch % 2]).wait()

            def inner(i, cur, ch=ch):
                kv = idxbuf[ch % 2, pl.ds(pl.multiple_of(i * 16, 16), 16)]
                m = (kv >= lo) & (kv < hi)
                posv = ch * chunk_len + i * 16 + lane
                pf = plsc.cumsum(m.astype(jnp.int32))
                dst = cur + pf - 1
                sel = m & (dst < _CAP_E)
                dstc = jnp.where(sel, dst, 0)
                packed = jnp.left_shift(kv - lo, 16) | posv
                plsc.store_scatter(e_pack, [dstc], packed, mask=sel)
                do_overflow(kv, posv, m & (dst >= _CAP_E))
                return cur + pf[15]

            cursor = lax.fori_loop(0, chunk_len // 16, inner, cursor)

        ec = jnp.minimum(cursor, _CAP_E)
        nv = (ec + 15) // 16

        def rescan(sp):
            base = jnp.bitwise_and(sp, 1) * _CAP_B

            def rs(j, c2):
                pk = e_pack[pl.ds(pl.multiple_of(j * 16, 16), 16)]
                m = ((j * 16 + lane) < ec) & (jnp.right_shift(pk, 19) == sp)
                pf = plsc.cumsum(m.astype(jnp.int32))
                dst = jnp.where(m, base + c2 + pf - 1, 0)
                plsc.store_scatter(lbuf, [dst], pk, mask=m)
                return c2 + pf[15]

            return lax.fori_loop(0, nv, rs, jnp.int32(0))

        c0 = rescan(jnp.int32(0))

        def subspan(s, carry):
            c_cur, c_prev = carry
            drain_outs(c_prev)
            pltpu.async_copy(
                table_hbm.at[pl.ds(lo + s * _R, _R)], rows, sem_row)
            c_next = rescan(jnp.minimum(s + 1, n_sub - 1))
            wait_rows()
            base_cur = jnp.bitwise_and(s, 1) * _CAP_B

            def issue(e, c):
                pk = lane_at(
                    lbuf[pl.ds(pl.multiple_of(
                        base_cur + jnp.right_shift(e, 4) * 16, 8), 16)],
                    jnp.bitwise_and(e, 15))
                pos = jnp.bitwise_and(pk, (1 << 16) - 1)
                row = jnp.bitwise_and(jnp.right_shift(pk, 16), _R - 1)
                pltpu.async_copy(rows.at[row], out_hbm.at[pos], sem_out)
                return c

            lax.fori_loop(0, c_cur, issue, 0)
            return (c_next, c_cur)

        _, c_last = lax.fori_loop(
            0, n_sub, subspan, (c0, jnp.int32(0)))
        drain_outs(c_last)

    return body


def kernel(idx, table):
    b, t = idx.shape
    v, d = table.shape
    out = _build(b * t, v, d)(idx.reshape(-1).astype(jnp.int32), table)
    return out.reshape(b, t, d)
